# Initial kernel scaffold; baseline (speedup 1.0000x reference)
import jax
import jax.numpy as jnp
from jax.experimental import pallas as pl


def kernel(table, idx):
    raise NotImplementedError("write your pallas kernel here")


# speedup vs baseline: 1.4373x; 1.4373x over previous
import functools

import jax
import jax.numpy as jnp
from jax import lax
from jax.experimental import pallas as pl
from jax.experimental.pallas import tpu as pltpu
from jax.experimental.pallas import tpu_sc as plsc

NC = 2
NS = 16
NW = NC * NS
IPG = 128


@functools.partial(jax.jit, static_argnames=("n_rows", "d"))
def _gather(table, idx_rows, n_rows, d):
    g_total = idx_rows.shape[0]
    gpw = g_total // NW

    mesh = plsc.VectorSubcoreMesh(core_axis_name="c", subcore_axis_name="s")

    @functools.partial(
        pl.kernel,
        out_type=jax.ShapeDtypeStruct((n_rows, d), jnp.float32),
        mesh=mesh,
        scratch_types=[
            pltpu.VMEM((gpw, IPG), jnp.int32),
            pltpu.VMEM((IPG, d), jnp.float32),
            pltpu.SemaphoreType.DMA,
        ],
        compiler_params=pltpu.CompilerParams(use_tc_tiling_on_sc=False),
    )
    def k(table_hbm, idx_hbm, out_hbm, idx_v, rows_v, sem):
        wid = lax.axis_index("s") * NC + lax.axis_index("c")
        gbase = wid * gpw
        pltpu.sync_copy(idx_hbm.at[pl.ds(gbase, gpw)], idx_v)

        def step(g, carry):
            pltpu.async_copy(table_hbm.at[idx_v.at[g]], rows_v, sem).wait()
            pltpu.sync_copy(rows_v, out_hbm.at[pl.ds((gbase + g) * IPG, IPG)])
            return carry

        lax.fori_loop(0, gpw, step, 0)

    return k(table, idx_rows)


def kernel(table, idx):
    n_rows = idx.size
    d = table.shape[1]
    idx_rows = idx.reshape(n_rows // IPG, IPG)
    out = _gather(table, idx_rows, n_rows, d)
    return out.reshape(idx.shape + (d,))


# speedup vs baseline: 1.5833x; 1.1016x over previous
import functools

import jax
import jax.numpy as jnp
from jax import lax
from jax.experimental import pallas as pl
from jax.experimental.pallas import tpu as pltpu
from jax.experimental.pallas import tpu_sc as plsc

NC = 2
NS = 16
NW = NC * NS
IPG = 128


@functools.partial(jax.jit, static_argnames=("n_rows", "d"))
def _gather(table, idx_rows, n_rows, d):
    g_total = idx_rows.shape[0]
    gpw = g_total // NW

    mesh = plsc.VectorSubcoreMesh(core_axis_name="c", subcore_axis_name="s")

    nbuf = 8
    assert gpw % nbuf == 0

    @functools.partial(
        pl.kernel,
        out_type=jax.ShapeDtypeStruct((n_rows, d), jnp.float32),
        mesh=mesh,
        scratch_types=[
            pltpu.VMEM((gpw, IPG), jnp.int32),
            pltpu.VMEM((nbuf, IPG, d), jnp.float32),
            pltpu.SemaphoreType.DMA((nbuf,)),
        ],
        compiler_params=pltpu.CompilerParams(use_tc_tiling_on_sc=False),
    )
    def k(table_hbm, idx_hbm, out_hbm, idx_v, rows_v, sems):
        wid = lax.axis_index("s") * NC + lax.axis_index("c")
        gbase = wid * gpw
        pltpu.sync_copy(idx_hbm.at[pl.ds(gbase, gpw)], idx_v)

        def gather(g, b):
            return pltpu.make_async_copy(
                table_hbm.at[idx_v.at[g]], rows_v.at[b], sems.at[b]
            )

        for b in range(nbuf):
            gather(b, b).start()

        @pl.loop(0, gpw, step=nbuf)
        def outer(i):
            for b in range(nbuf):
                g = i + b
                gather(g, b).wait()
                pltpu.sync_copy(
                    rows_v.at[b], out_hbm.at[pl.ds((gbase + g) * IPG, IPG)]
                )

                @pl.when(g + nbuf < gpw)
                def _():
                    gather(g + nbuf, b).start()

    return k(table, idx_rows)


def kernel(table, idx):
    n_rows = idx.size
    d = table.shape[1]
    idx_rows = idx.reshape(n_rows // IPG, IPG)
    out = _gather(table, idx_rows, n_rows, d)
    return out.reshape(idx.shape + (d,))
